# baseline (speedup 1.0000x reference)
import functools

import jax
import jax.numpy as jnp
from jax import lax
from jax.experimental import pallas as pl
from jax.experimental.pallas import tpu as pltpu
from jax.experimental.pallas import tpu_sc as plsc

NC = 2
NS = 16
NW = NC * NS
L = 16
CHUNK = 2000
SPLIT_B = 4096


def _unpack(p):
    x = plsc.bitcast(lax.shift_left(p, 16), jnp.float32)
    y = plsc.bitcast(lax.bitwise_and(p, jnp.int32(-65536)), jnp.float32)
    return x, y


def _tc_split_body(in_ref, o0, o1, o2):
    o0[...] = in_ref[0, :]
    o1[...] = in_ref[1, :]
    o2[...] = in_ref[2, :]


def _split_columns(trips_t):
    T = trips_t.shape[1]
    grid = (T + SPLIT_B - 1) // SPLIT_B
    out = jax.ShapeDtypeStruct((T,), jnp.int32)
    return pl.pallas_call(
        _tc_split_body,
        grid=(grid,),
        in_specs=[pl.BlockSpec((3, SPLIT_B), lambda i: (0, i))],
        out_specs=[pl.BlockSpec((SPLIT_B,), lambda i: (i,))] * 3,
        out_shape=[out, out, out],
    )(trips_t)


def _sc_body(n_chunks, i_hbm, j_hbm, k_hbm, w_hbm, table_hbm,
             loss_out, viol_out, table_v,
             i_v0, j_v0, k_v0, w_v0, i_v1, j_v1, k_v1, w_v1,
             stage_v, sem0, sem1):
    c = lax.axis_index("c")
    s = lax.axis_index("s")
    wid = s * NC + c

    pltpu.sync_copy(table_hbm, table_v)

    zero = jnp.zeros((L,), jnp.float32)
    bufs = ((i_v0, j_v0, k_v0, w_v0, sem0), (i_v1, j_v1, k_v1, w_v1, sem1))
    n_mine = (n_chunks - wid + NW - 1) // NW

    def _descs(t, b):
        g = wid + t * NW
        sl = pl.ds(g * CHUNK, CHUNK)
        iv, jv, kv, wv, sem = bufs[b]
        return (
            pltpu.make_async_copy(i_hbm.at[sl], iv, sem),
            pltpu.make_async_copy(j_hbm.at[sl], jv, sem),
            pltpu.make_async_copy(k_hbm.at[sl], kv, sem),
            pltpu.make_async_copy(w_hbm.at[sl], wv, sem),
        )

    def _start(t, b):
        @pl.when(t < n_mine)
        def _():
            for d in _descs(t, b):
                d.start()

    def _compute(t, b, carry):
        iv, jv, kv, wv, _ = bufs[b]
        for d in _descs(t, b):
            d.wait()

        def vec_body(v, c2):
            lv, vv = c2
            sl = pl.ds(v * L, L)
            xi, yi = _unpack(plsc.load_gather(table_v, [iv[sl]]))
            xj, yj = _unpack(plsc.load_gather(table_v, [jv[sl]]))
            xk, yk = _unpack(plsc.load_gather(table_v, [kv[sl]]))
            dxij = xi - xj
            dyij = yi - yj
            dxik = xi - xk
            dyik = yi - yk
            dij = 1.0 + dxij * dxij + dyij * dyij
            dik = 1.0 + dxik * dxik + dyik * dyik
            w = wv[sl]
            lv = lv + w * (dij / (dij + dik))
            vv = vv + jnp.where(dij > dik, 1.0, 0.0).astype(jnp.float32)
            return lv, vv

        return lax.fori_loop(0, CHUNK // L, vec_body, carry)

    _start(0, 0)

    def pair_body(u, carry):
        t0 = 2 * u
        _start(t0 + 1, 1)
        carry = lax.cond(t0 < n_mine,
                         lambda cc: _compute(t0, 0, cc), lambda cc: cc, carry)
        _start(t0 + 2, 0)
        carry = lax.cond(t0 + 1 < n_mine,
                         lambda cc: _compute(t0 + 1, 1, cc), lambda cc: cc,
                         carry)
        return carry

    lv, vv = lax.fori_loop(0, (n_mine + 1) // 2, pair_body, (zero, zero))

    stage_v[...] = lv
    pltpu.sync_copy(stage_v, loss_out.at[wid])
    stage_v[...] = vv
    pltpu.sync_copy(stage_v, viol_out.at[wid])


def kernel(embed_init, triplets, weights):
    n = embed_init.shape[0]
    T = triplets.shape[0]

    b16 = lax.bitcast_convert_type(embed_init.astype(jnp.bfloat16), jnp.uint16)
    b32 = b16.astype(jnp.uint32)
    packed = lax.bitcast_convert_type(b32[:, 0] | (b32[:, 1] << 16), jnp.int32)

    trips = triplets.astype(jnp.int32)
    w = weights.astype(jnp.float32)
    pad = (-T) % CHUNK
    if pad:
        trips = jnp.concatenate([trips, jnp.zeros((pad, 3), jnp.int32)])
        w = jnp.concatenate([w, jnp.zeros((pad,), jnp.float32)])
    n_chunks = (T + pad) // CHUNK

    ti, tj, tk = _split_columns(trips.T)

    mesh = plsc.VectorSubcoreMesh(
        core_axis_name="c", subcore_axis_name="s", num_cores=NC, num_subcores=NS
    )
    fn = pl.kernel(
        functools.partial(_sc_body, n_chunks),
        out_type=(
            jax.ShapeDtypeStruct((NW, L), jnp.float32),
            jax.ShapeDtypeStruct((NW, L), jnp.float32),
        ),
        mesh=mesh,
        scratch_types=[
            pltpu.VMEM((n,), jnp.int32),
            pltpu.VMEM((CHUNK,), jnp.int32),
            pltpu.VMEM((CHUNK,), jnp.int32),
            pltpu.VMEM((CHUNK,), jnp.int32),
            pltpu.VMEM((CHUNK,), jnp.float32),
            pltpu.VMEM((CHUNK,), jnp.int32),
            pltpu.VMEM((CHUNK,), jnp.int32),
            pltpu.VMEM((CHUNK,), jnp.int32),
            pltpu.VMEM((CHUNK,), jnp.float32),
            pltpu.VMEM((L,), jnp.float32),
            pltpu.SemaphoreType.DMA,
            pltpu.SemaphoreType.DMA,
        ],
        compiler_params=pltpu.CompilerParams(needs_layout_passes=False),
    )
    loss_parts, viol_parts = fn(ti, tj, tk, w, packed)
    return jnp.sum(loss_parts), jnp.sum(viol_parts)


# speedup vs baseline: 3.1854x; 3.1854x over previous
import functools

import jax
import jax.numpy as jnp
from jax import lax
from jax.experimental import pallas as pl
from jax.experimental.pallas import tpu as pltpu
from jax.experimental.pallas import tpu_sc as plsc

NC = 2
NS = 16
NW = NC * NS
L = 16
CHUNK = 2688


def _unpack(p):
    x = plsc.bitcast(lax.shift_left(p, 16), jnp.float32)
    y = plsc.bitcast(lax.bitwise_and(p, jnp.int32(-65536)), jnp.float32)
    return x, y


def _trimap_vec(iv_sl, jv_sl, kv_sl, w_sl, table_v, lv, vv):
    xi, yi = _unpack(plsc.load_gather(table_v, [iv_sl]))
    xj, yj = _unpack(plsc.load_gather(table_v, [jv_sl]))
    xk, yk = _unpack(plsc.load_gather(table_v, [kv_sl]))
    dxij = xi - xj
    dyij = yi - yj
    dxik = xi - xk
    dyik = yi - yk
    dij = 1.0 + dxij * dxij + dyij * dyij
    dik = 1.0 + dxik * dxik + dyik * dyik
    lv = lv + w_sl * (dij / (dij + dik))
    vv = vv + jnp.where(dij > dik, 1.0, 0.0).astype(jnp.float32)
    return lv, vv


def _sc_body(n_chunks, tail_pad, trip_hbm, tail_hbm, wtail_hbm, w_hbm,
             table_hbm, loss_out, viol_out, table_v,
             t_v0, w_v0, t_v1, w_v1, tail_v, wt_v, stage_v, sem0, sem1):
    c = lax.axis_index("c")
    s = lax.axis_index("s")
    wid = s * NC + c

    pltpu.sync_copy(table_hbm, table_v)

    zero = jnp.zeros((L,), jnp.float32)
    bufs = ((t_v0, w_v0, sem0), (t_v1, w_v1, sem1))
    n_mine = (n_chunks - wid + NW - 1) // NW

    def _descs(t, b):
        g = wid + t * NW
        tv, wv, sem = bufs[b]
        return (
            pltpu.make_async_copy(
                trip_hbm.at[:, pl.ds(g * CHUNK, CHUNK)], tv, sem),
            pltpu.make_async_copy(w_hbm.at[pl.ds(g * CHUNK, CHUNK)], wv, sem),
        )

    def _start(t, b):
        @pl.when(t < n_mine)
        def _():
            for d in _descs(t, b):
                d.start()

    def _compute(t, b, carry):
        tv, wv, _ = bufs[b]
        for d in _descs(t, b):
            d.wait()

        def vec_body(v, c2):
            lv, vv = c2
            sl = pl.ds(v * L, L)
            return _trimap_vec(tv[0, sl], tv[1, sl], tv[2, sl], wv[sl],
                               table_v, lv, vv)

        return lax.fori_loop(0, CHUNK // L, vec_body, carry)

    _start(0, 0)

    def pair_body(u, carry):
        t0 = 2 * u
        _start(t0 + 1, 1)
        carry = lax.cond(t0 < n_mine,
                         lambda cc: _compute(t0, 0, cc), lambda cc: cc, carry)
        _start(t0 + 2, 0)
        carry = lax.cond(t0 + 1 < n_mine,
                         lambda cc: _compute(t0 + 1, 1, cc), lambda cc: cc,
                         carry)
        return carry

    lv, vv = lax.fori_loop(0, (n_mine + 1) // 2, pair_body, (zero, zero))

    if tail_pad:
        @pl.when(wid == 0)
        def _():
            pltpu.sync_copy(tail_hbm, tail_v)
            pltpu.sync_copy(wtail_hbm, wt_v)

        def tail_body(v, c2):
            lv2, vv2 = c2
            sl = pl.ds(v * L, L)
            sl_j = pl.ds(tail_pad + v * L, L)
            sl_k = pl.ds(2 * tail_pad + v * L, L)
            return _trimap_vec(tail_v[sl], tail_v[sl_j], tail_v[sl_k],
                               wt_v[sl], table_v, lv2, vv2)

        lv, vv = lax.cond(
            wid == 0,
            lambda cc: lax.fori_loop(0, tail_pad // L, tail_body, cc),
            lambda cc: cc, (lv, vv))

    stage_v[...] = lv
    pltpu.sync_copy(stage_v, loss_out.at[wid])
    stage_v[...] = vv
    pltpu.sync_copy(stage_v, viol_out.at[wid])


def kernel(embed_init, triplets, weights):
    n = embed_init.shape[0]
    T = triplets.shape[0]

    b16 = lax.bitcast_convert_type(embed_init.astype(jnp.bfloat16), jnp.uint16)
    b32 = b16.astype(jnp.uint32)
    packed = lax.bitcast_convert_type(b32[:, 0] | (b32[:, 1] << 16), jnp.int32)

    trips = triplets.astype(jnp.int32)
    w = weights.astype(jnp.float32)

    main = (T // CHUNK) * CHUNK
    n_chunks = main // CHUNK
    tail_n = T - main
    tail_pad = (tail_n + L - 1) // L * L

    trips_t = trips.T

    if tail_pad:
        tt = jnp.zeros((tail_pad, 3), jnp.int32).at[:tail_n].set(trips[main:])
        tail_ijk = tt.T.reshape(-1)
        w_tail = jnp.zeros((tail_pad,), jnp.float32).at[:tail_n].set(w[main:])
    else:
        tail_ijk = jnp.zeros((3 * L,), jnp.int32)
        w_tail = jnp.zeros((L,), jnp.float32)
        tail_pad = 0

    tail_buf = max(tail_pad, L)

    mesh = plsc.VectorSubcoreMesh(
        core_axis_name="c", subcore_axis_name="s", num_cores=NC, num_subcores=NS
    )
    fn = pl.kernel(
        functools.partial(_sc_body, n_chunks, tail_pad),
        out_type=(
            jax.ShapeDtypeStruct((NW, L), jnp.float32),
            jax.ShapeDtypeStruct((NW, L), jnp.float32),
        ),
        mesh=mesh,
        scratch_types=[
            pltpu.VMEM((n,), jnp.int32),
            pltpu.VMEM((3, CHUNK), jnp.int32),
            pltpu.VMEM((CHUNK,), jnp.float32),
            pltpu.VMEM((3, CHUNK), jnp.int32),
            pltpu.VMEM((CHUNK,), jnp.float32),
            pltpu.VMEM((3 * tail_buf,), jnp.int32),
            pltpu.VMEM((tail_buf,), jnp.float32),
            pltpu.VMEM((L,), jnp.float32),
            pltpu.SemaphoreType.DMA,
            pltpu.SemaphoreType.DMA,
        ],
        compiler_params=pltpu.CompilerParams(
            needs_layout_passes=False, use_tc_tiling_on_sc=True),
    )
    loss_parts, viol_parts = fn(trips_t, tail_ijk, w_tail, w, packed)
    return jnp.sum(loss_parts), jnp.sum(viol_parts)
